# baseline (speedup 1.0000x reference)
import functools

import jax
import jax.numpy as jnp
from jax import lax
from jax.experimental import pallas as pl
from jax.experimental.pallas import tpu as pltpu
from jax.experimental.pallas import tpu_sc as plsc

_PACK_BLOCK = 1024
_GATHER_WINDOW = 128


def _tc_transpose_pack(table_t):
    d, v = table_t.shape
    nblk = (v + _PACK_BLOCK - 1) // _PACK_BLOCK
    v_pad = nblk * _PACK_BLOCK

    def body(x_ref, o_ref):
        t = jnp.transpose(x_ref[...], (1, 0))
        h = _PACK_BLOCK // 2
        o_ref[...] = jnp.concatenate([t[:h, :], t[h:, :]], axis=1)

    return pl.pallas_call(
        body,
        grid=(nblk,),
        in_specs=[pl.BlockSpec((d, _PACK_BLOCK), lambda i: (0, i))],
        out_specs=pl.BlockSpec((_PACK_BLOCK // 2, 2 * d), lambda i: (i, 0)),
        out_shape=jax.ShapeDtypeStruct((v_pad // 2, 2 * d), table_t.dtype),
    )(table_t)


def _sc_gather(table, idx2):
    n = idx2.shape[0] * idx2.shape[1]
    _, d = table.shape
    mesh = plsc.VectorSubcoreMesh(core_axis_name="c", subcore_axis_name="s")

    @functools.partial(
        pl.kernel,
        out_type=jax.ShapeDtypeStruct((n, d), table.dtype),
        mesh=mesh,
        compiler_params=pltpu.CompilerParams(use_tc_tiling_on_sc=False),
    )
    def gather_kernel(tab_hbm, i_hbm, o_hbm):
        def body(i_vmem, o_vmem):
            pltpu.sync_copy(tab_hbm.at[i_vmem.at[0]], o_vmem)

        pltpu.emit_pipeline(
            body,
            grid=(n // _GATHER_WINDOW,),
            in_specs=[pl.BlockSpec((1, _GATHER_WINDOW), lambda i: (i, 0))],
            out_specs=[pl.BlockSpec((_GATHER_WINDOW, d), lambda i: (i, 0))],
            core_axis_name=("c", "s"),
            dimension_semantics=(pltpu.PARALLEL,),
        )(i_hbm, o_hbm)

    return gather_kernel(table, idx2)


def _tc_project_t(rows_packed, w, b_tile, hist, batch):
    _, dd = rows_packed.shape
    d = dd // 2
    p = w.shape[0]
    half = batch // 2

    def body(x_ref, w_ref, b_ref, o_ref):
        x = x_ref[...]
        bcol = b_ref[:, 0:1]
        dims = (((1,), (1,)), ((), ()))
        ye = lax.dot_general(
            w_ref[...], x[:, :d], dims,
            preferred_element_type=jnp.float32,
            precision=lax.Precision.HIGHEST,
        )
        yo = lax.dot_general(
            w_ref[...], x[:, d:], dims,
            preferred_element_type=jnp.float32,
            precision=lax.Precision.HIGHEST,
        )
        o_ref[0, :, :half] = ye + bcol
        o_ref[0, :, half:] = yo + bcol

    return pl.pallas_call(
        body,
        grid=(hist,),
        in_specs=[
            pl.BlockSpec((half, dd), lambda i: (i, 0)),
            pl.BlockSpec((p, d), lambda i: (0, 0)),
            pl.BlockSpec((p, 128), lambda i: (0, 0)),
        ],
        out_specs=pl.BlockSpec((1, p, batch), lambda i: (i, 0, 0)),
        out_shape=jax.ShapeDtypeStruct((hist, p, batch), jnp.float32),
    )(rows_packed, w, b_tile)


def kernel(inputs, weights, W, b):
    batch, hist = inputs.shape
    embed = weights.shape[1]
    proj = W.shape[0]
    n = batch * hist
    half = batch // 2

    tab_packed = _tc_transpose_pack(weights.T)
    tab_rows = tab_packed.reshape(2 * tab_packed.shape[0], embed)

    idx_t = inputs.T
    r = idx_t % _PACK_BLOCK
    idx_view = (idx_t - r) + 2 * (r % (_PACK_BLOCK // 2)) + r // (_PACK_BLOCK // 2)
    idx_perm = jnp.stack([idx_view[:, :half], idx_view[:, half:]], axis=2)
    idx_rows = idx_perm.reshape(n // _GATHER_WINDOW, _GATHER_WINDOW)

    emb = _sc_gather(tab_rows, idx_rows)
    emb_packed = emb.reshape(n // 2, 2 * embed)

    b_tile = jnp.broadcast_to(b[:, None], (proj, 128))
    out_t = _tc_project_t(emb_packed, W, b_tile, hist, batch)
    return jnp.transpose(out_t, (2, 0, 1))


# speedup vs baseline: 1.7030x; 1.7030x over previous
import functools

import jax
import jax.numpy as jnp
from jax import lax
from jax.experimental import pallas as pl
from jax.experimental.pallas import tpu as pltpu
from jax.experimental.pallas import tpu_sc as plsc

_PACK_BLOCK = 4096
_GATHER_WINDOW = 128


def _tc_transpose_pack(table_t):
    d, v = table_t.shape
    nblk = (v + _PACK_BLOCK - 1) // _PACK_BLOCK
    v_pad = nblk * _PACK_BLOCK

    def body(x_ref, o_ref):
        t = jnp.transpose(x_ref[...], (1, 0))
        h = _PACK_BLOCK // 2
        o_ref[...] = jnp.concatenate([t[:h, :], t[h:, :]], axis=1)

    return pl.pallas_call(
        body,
        grid=(nblk,),
        in_specs=[pl.BlockSpec((d, _PACK_BLOCK), lambda i: (0, i))],
        out_specs=pl.BlockSpec((_PACK_BLOCK // 2, 2 * d), lambda i: (i, 0)),
        out_shape=jax.ShapeDtypeStruct((v_pad // 2, 2 * d), table_t.dtype),
    )(table_t)


def _sc_gather(table, idx2):
    n = idx2.shape[0] * idx2.shape[1]
    _, d = table.shape
    mesh = plsc.VectorSubcoreMesh(core_axis_name="c", subcore_axis_name="s")

    @functools.partial(
        pl.kernel,
        out_type=jax.ShapeDtypeStruct((n, d), table.dtype),
        mesh=mesh,
        compiler_params=pltpu.CompilerParams(use_tc_tiling_on_sc=False),
    )
    def gather_kernel(tab_hbm, i_hbm, o_hbm):
        def body(i_vmem, o_vmem):
            pltpu.sync_copy(tab_hbm.at[i_vmem.at[0]], o_vmem)

        pltpu.emit_pipeline(
            body,
            grid=(n // _GATHER_WINDOW,),
            in_specs=[pl.BlockSpec((1, _GATHER_WINDOW), lambda i: (i, 0))],
            out_specs=[pl.BlockSpec((_GATHER_WINDOW, d), lambda i: (i, 0))],
            core_axis_name=("c", "s"),
            dimension_semantics=(pltpu.PARALLEL,),
        )(i_hbm, o_hbm)

    return gather_kernel(table, idx2)


def _tc_project_t(rows_packed, w, b_tile, hist, batch):
    _, dd = rows_packed.shape
    d = dd // 2
    p = w.shape[0]
    half = batch // 2

    def body(x_ref, w_ref, b_ref, o_ref):
        x = x_ref[...]
        bcol = b_ref[:, 0:1]
        dims = (((1,), (1,)), ((), ()))
        ye = lax.dot_general(
            w_ref[...], x[:, :d], dims,
            preferred_element_type=jnp.float32,
            precision=lax.Precision.DEFAULT,
        )
        yo = lax.dot_general(
            w_ref[...], x[:, d:], dims,
            preferred_element_type=jnp.float32,
            precision=lax.Precision.DEFAULT,
        )
        o_ref[0, :, :half] = ye + bcol
        o_ref[0, :, half:] = yo + bcol

    return pl.pallas_call(
        body,
        grid=(hist,),
        in_specs=[
            pl.BlockSpec((half, dd), lambda i: (i, 0)),
            pl.BlockSpec((p, d), lambda i: (0, 0)),
            pl.BlockSpec((p, 128), lambda i: (0, 0)),
        ],
        out_specs=pl.BlockSpec((1, p, batch), lambda i: (i, 0, 0)),
        out_shape=jax.ShapeDtypeStruct((hist, p, batch), jnp.float32),
    )(rows_packed, w, b_tile)


def kernel(inputs, weights, W, b):
    batch, hist = inputs.shape
    embed = weights.shape[1]
    proj = W.shape[0]
    n = batch * hist
    half = batch // 2

    tab_packed = _tc_transpose_pack(weights.T)
    tab_rows = tab_packed.reshape(2 * tab_packed.shape[0], embed)

    idx_t = inputs.T
    r = idx_t % _PACK_BLOCK
    idx_view = (idx_t - r) + 2 * (r % (_PACK_BLOCK // 2)) + r // (_PACK_BLOCK // 2)
    idx_perm = jnp.stack([idx_view[:, :half], idx_view[:, half:]], axis=2)
    idx_rows = idx_perm.reshape(n // _GATHER_WINDOW, _GATHER_WINDOW)

    emb = _sc_gather(tab_rows, idx_rows)
    emb_packed = emb.reshape(n // 2, 2 * embed)

    b_tile = jnp.broadcast_to(b[:, None], (proj, 128))
    out_t = _tc_project_t(emb_packed, W, b_tile, hist, batch)
    return jnp.transpose(out_t, (2, 0, 1))
